# baseline (speedup 1.0000x reference)
import functools
import math

import jax
import jax.numpy as jnp
import numpy as np
from jax import lax
from jax.experimental import pallas as pl
from jax.experimental.pallas import tpu as pltpu
from jax.experimental.pallas import tpu_sc as plsc

D_MODEL = 1024
MAX_LEN = 5000
PE_DIM = D_MODEL // 4


def _pe_table() -> np.ndarray:
    position = np.arange(MAX_LEN, dtype=np.float32)[:, None]
    div_term = np.exp(
        np.arange(0, PE_DIM, 2).astype(np.float32) * (-math.log(10000.0) / PE_DIM)
    )
    pe = np.zeros((MAX_LEN, PE_DIM), dtype=np.float32)
    pe[:, 0::2] = np.sin(position * div_term)
    pe[:, 1::2] = np.cos(position * div_term)
    return pe.reshape(2 * MAX_LEN, 128)


_PE128 = _pe_table()

_CHUNK = 384


def _sc_gather(pe, idx, n_rows):
    info = plsc.get_sparse_core_info()
    nc, ns = info.num_cores, info.num_subcores
    nw = nc * ns
    rows_per_w = n_rows // nw
    assert rows_per_w * nw == n_rows and rows_per_w % _CHUNK == 0
    n_chunks = rows_per_w // _CHUNK

    mesh = plsc.VectorSubcoreMesh(core_axis_name="c", subcore_axis_name="s")

    @functools.partial(
        pl.kernel,
        mesh=mesh,
        out_type=jax.ShapeDtypeStruct((n_rows, 128), jnp.float32),
        scratch_types=[
            pltpu.VMEM((_CHUNK,), jnp.int32),
            pltpu.VMEM((_CHUNK,), jnp.int32),
            pltpu.VMEM((_CHUNK, 128), jnp.float32),
            pltpu.VMEM((_CHUNK, 128), jnp.float32),
            pltpu.SemaphoreType.DMA,
            pltpu.SemaphoreType.DMA,
            pltpu.SemaphoreType.DMA,
        ],
    )
    def k(pe_hbm, idx_hbm, out_hbm, idx0, idx1, rows0, rows1, sg, sw0, sw1):
        wid = lax.axis_index("s") * nc + lax.axis_index("c")
        w_base = wid * rows_per_w
        idx_v = (idx0, idx1)
        rows_v = (rows0, rows1)
        sw = (sw0, sw1)
        writes = [None, None]
        for u in range(n_chunks):
            p = u % 2
            if writes[p] is not None:
                writes[p].wait()
            base = w_base + u * _CHUNK
            pltpu.sync_copy(idx_hbm.at[pl.ds(base, _CHUNK)], idx_v[p])
            pltpu.async_copy(pe_hbm.at[idx_v[p]], rows_v[p], sg).wait()
            writes[p] = pltpu.async_copy(
                rows_v[p], out_hbm.at[pl.ds(base, _CHUNK)], sw[p]
            )
        for w in writes:
            if w is not None:
                w.wait()

    return k(pe, idx)


def _tc_add(seq, emb128, s_pad, cb):
    b, s, d = seq.shape
    n_tau = s_pad // (8 * cb)
    spec_sd = pl.BlockSpec((1, 8 * cb, d), lambda i, t: (i, t, 0))

    def body(seq_ref, emb_ref, out_ref, embout_ref):
        e = emb_ref[...].reshape(1, 8 * cb, d)
        out_ref[...] = seq_ref[...] + e
        embout_ref[...] = e

    return pl.pallas_call(
        body,
        grid=(b, n_tau),
        in_specs=[
            spec_sd,
            pl.BlockSpec(
                (None, 64 * cb, 128), lambda i, t: (i * n_tau + t, 0, 0)
            ),
        ],
        out_specs=[spec_sd, spec_sd],
        out_shape=[
            jax.ShapeDtypeStruct((b, s, d), jnp.float32),
            jax.ShapeDtypeStruct((b, s, d), jnp.float32),
        ],
    )(seq, emb128.reshape(b * n_tau, 64 * cb, 128))


def kernel(seq, coords, seq_id):
    b, s, d = seq.shape
    s_pad = (s + 7) // 8 * 8

    ii = jnp.concatenate(
        [coords.astype(jnp.int32), seq_id[..., None].astype(jnp.int32)], axis=-1
    )
    ii = jnp.clip(ii, 0, MAX_LEN - 1)
    ii = jnp.pad(ii, ((0, 0), (1, s_pad - s), (0, 0)))
    jj = (2 * ii)[..., None] + jnp.arange(2, dtype=jnp.int32)
    idx = jj.reshape(b * s_pad * 8)

    pe = jnp.asarray(_PE128)
    emb128 = _sc_gather(pe, idx, b * s_pad * 8)
    out, emb = _tc_add(seq, emb128, s_pad, cb=11)
    return (out, emb)
